# baseline (speedup 1.0000x reference)
import functools

import jax
import jax.numpy as jnp
from jax import lax
from jax.experimental import pallas as pl
from jax.experimental.pallas import tpu as pltpu
from jax.experimental.pallas import tpu_sc as plsc

N_FIELDS = 26
VOCAB = 100001
DIM = 32
BATCH = 4096
LANES = 16

NUM_CORES = 2
NUM_SUBCORES = 16
NUM_WORKERS = NUM_CORES * NUM_SUBCORES

NTC = (VOCAB + 127) // 128
TILE = 1024
NSTRIP = DIM // 8
NSTRIPS_ALL = N_FIELDS * NSTRIP
IMG = N_FIELDS * NSTRIP * NTC * TILE
STRIDE_STRIP = NTC * TILE
NHALF = 2
HB = BATCH // NHALF
UNIT = 8 * HB


def _tile_copy(tab_t):
    mesh = plsc.VectorSubcoreMesh(core_axis_name="c", subcore_axis_name="s")

    @functools.partial(
        pl.kernel,
        mesh=mesh,
        out_type=jax.ShapeDtypeStruct(
            (N_FIELDS, NSTRIP, NTC, 8, 128), jnp.float32),
        scratch_types=[pltpu.SemaphoreType.DMA],
    )
    def k(tab_hbm, img_hbm, sem):
        wid = lax.axis_index("s") * NUM_CORES + lax.axis_index("c")

        def strip_body(i, _):
            s = wid + i * NUM_WORKERS

            @pl.when(s < NSTRIPS_ALL)
            def _():
                f = s // NSTRIP
                tr = s % NSTRIP

                def gbody(g, _):
                    pltpu.async_copy(
                        tab_hbm.at[f, pl.ds(tr * 8, 8),
                                   pl.ds(g * 128, 128)],
                        img_hbm.at[f, tr, g], sem)
                    return 0

                lax.fori_loop(0, NTC, gbody, 0)

                def dbody(g, _):
                    pltpu.make_async_copy(
                        tab_hbm.at[0, pl.ds(0, 8), pl.ds(0, 128)],
                        img_hbm.at[0, 0, 0], sem).wait()
                    return 0

                lax.fori_loop(0, NTC, dbody, 0)

            return 0

        lax.fori_loop(0, (NSTRIPS_ALL + NUM_WORKERS - 1) // NUM_WORKERS,
                      strip_body, 0)

    return k(tab_t)


def _gather(img_flat, idx_p):
    mesh = plsc.VectorSubcoreMesh(core_axis_name="c", subcore_axis_name="s")

    @functools.partial(
        pl.kernel,
        mesh=mesh,
        out_type=jax.ShapeDtypeStruct((N_FIELDS, DIM, BATCH), jnp.float32),
        scratch_types=[
            pltpu.VMEM((8, BATCH), jnp.int32),
            pltpu.VMEM((BATCH,), jnp.int32),
            pltpu.VMEM((UNIT,), jnp.int32),
            pltpu.VMEM((UNIT,), jnp.float32),
            pltpu.SemaphoreType.DMA,
            pltpu.SemaphoreType.DMA,
        ],
        compiler_params=pltpu.CompilerParams(use_tc_tiling_on_sc=False),
    )
    def k(img_hbm, idx_hbm, out_hbm, idxblk, base, pidx, gbuf, gsem, osem):
        wid = lax.axis_index("s") * NUM_CORES + lax.axis_index("c")

        @pl.when(wid < N_FIELDS)
        def _():
            f = wid
            rb8 = pl.multiple_of((f // 8) * 8, 8)
            fr = f % 8
            pltpu.sync_copy(idx_hbm.at[pl.ds(rb8, 8)], idxblk)

            def bbody(t, _):
                v = idxblk[fr, pl.ds(t * LANES, LANES)]
                base[pl.ds(t * LANES, LANES)] = ((v >> 7) * TILE + (v & 127))
                return 0

            lax.fori_loop(0, BATCH // LANES, bbody, 0)

            ocopies = []
            for tr in range(NSTRIP):
                for h in range(NHALF):
                    def pbody(t, _, tr=tr, h=h):
                        r = t // (HB // LANES)
                        j = t % (HB // LANES)
                        bs = base[pl.ds(h * HB + j * LANES, LANES)]
                        soff = (f * NSTRIP + tr) * STRIDE_STRIP + r * 128
                        pidx[pl.ds(r * HB + j * LANES, LANES)] = bs + soff
                        return 0

                    lax.fori_loop(0, 8 * (HB // LANES), pbody, 0)
                    if ocopies:
                        for cp in ocopies[-8:]:
                            cp.wait()
                    pltpu.async_copy(img_hbm.at[pidx], gbuf, gsem).wait()
                    for r in range(8):
                        ocopies.append(
                            pltpu.async_copy(
                                gbuf.at[pl.ds(r * HB, HB)],
                                out_hbm.at[f, tr * 8 + r, pl.ds(h * HB, HB)],
                                osem))
            for cp in ocopies[-8:]:
                cp.wait()

    return k(img_flat, idx_p)


def kernel(indices, tables):
    tab_t = jnp.transpose(tables, (0, 2, 1))
    idx_p = jnp.pad(indices.T, ((0, 6), (0, 0)))
    img = _tile_copy(tab_t)
    out_t = _gather(img.reshape(IMG), idx_p)
    return jnp.transpose(out_t, (2, 0, 1))


# speedup vs baseline: 2.5071x; 2.5071x over previous
import functools

import jax
import jax.numpy as jnp
from jax import lax
from jax.experimental import pallas as pl
from jax.experimental.pallas import tpu as pltpu
from jax.experimental.pallas import tpu_sc as plsc

N_FIELDS = 26
VOCAB = 100001
DIM = 32
BATCH = 4096

NUM_CORES = 2
NUM_SUBCORES = 16
NUM_WORKERS = NUM_CORES * NUM_SUBCORES
PAIRS = N_FIELDS * DIM
PPW = PAIRS // NUM_WORKERS


def kernel(indices, tables):
    tab_t = jnp.transpose(tables, (0, 2, 1))
    idx_t = indices.T

    mesh = plsc.VectorSubcoreMesh(core_axis_name="c", subcore_axis_name="s")

    @functools.partial(
        pl.kernel,
        mesh=mesh,
        out_type=jax.ShapeDtypeStruct((N_FIELDS, DIM, BATCH), jnp.float32),
        scratch_types=[
            pltpu.VMEM((2, BATCH), jnp.int32),
            pltpu.VMEM((PPW, BATCH), jnp.float32),
            pltpu.SemaphoreType.DMA,
            pltpu.SemaphoreType.DMA,
        ],
        compiler_params=pltpu.CompilerParams(use_tc_tiling_on_sc=False),
    )
    def k(idx_hbm, tab_hbm, out_hbm, idx_v, gbuf, gsem, osem):
        wid = lax.axis_index("s") * NUM_CORES + lax.axis_index("c")
        p0 = wid * PPW
        f0 = p0 // DIM
        pltpu.sync_copy(idx_hbm.at[f0], idx_v.at[0])
        f1 = jnp.minimum(f0 + 1, N_FIELDS - 1)
        pltpu.sync_copy(idx_hbm.at[f1], idx_v.at[1])
        gcopies = []
        for j in range(PPW):
            p = p0 + j
            f = p // DIM
            d = p % DIM
            gcopies.append(
                pltpu.async_copy(
                    tab_hbm.at[f, d].at[idx_v.at[f - f0]],
                    gbuf.at[j], gsem))
        for cp in gcopies:
            cp.wait()
        ocopies = []
        for j in range(PPW):
            p = p0 + j
            ocopies.append(
                pltpu.async_copy(
                    gbuf.at[j], out_hbm.at[p // DIM, p % DIM], osem))
        for cp in ocopies:
            cp.wait()

    out_t = k(idx_t, tab_t)
    return jnp.transpose(out_t, (2, 0, 1))
